# baseline (speedup 1.0000x reference)
import functools

import jax
import jax.numpy as jnp
from jax import lax
from jax.experimental import pallas as pl
from jax.experimental.pallas import tpu as pltpu
from jax.experimental.pallas import tpu_sc as plsc

N_NODES = 10000
N_EDGES = 320000
D = 128
DH = D // 2
N_CLASSES = 40

NC = 2
NS = 16
NW = NC * NS
CHUNK = 256
E_PAD = 327680
PAD_COL = 10200
NCH_DEG = E_PAD // (NW * CHUNK)
NCH_SCAT = E_PAD // (NS * CHUNK)
NPAD = 10240
RPT = NPAD // NS

_mesh = plsc.VectorSubcoreMesh(core_axis_name="c", subcore_axis_name="s")
_params = pltpu.CompilerParams(use_tc_tiling_on_sc=False)


def _deg_partials(col3, ones, zeros):
    @functools.partial(
        pl.kernel,
        mesh=_mesh,
        compiler_params=_params,
        out_type=jax.ShapeDtypeStruct((NC, NPAD, 16), jnp.float32),
        scratch_types=[
            pltpu.VMEM((NCH_DEG, CHUNK), jnp.int32),
            pltpu.VMEM((CHUNK, 16), jnp.float32),
            pltpu.VMEM_SHARED((NPAD, 16), jnp.float32),
            pltpu.SemaphoreType.DMA,
        ],
    )
    def k(col_hbm, ones_hbm, zeros_hbm, out_hbm, col_v, ones_v, acc_sh, sem):
        c = lax.axis_index("c")
        s = lax.axis_index("s")
        wid = c * NS + s
        pltpu.async_copy(zeros_hbm, acc_sh.at[pl.ds(s * RPT, RPT)], sem).wait()
        pltpu.async_copy(ones_hbm, ones_v, sem).wait()
        pltpu.async_copy(col_hbm.at[wid], col_v, sem).wait()
        plsc.subcore_barrier()

        @pl.loop(0, NCH_DEG)
        def _(j):
            pltpu.sync_copy(ones_v, acc_sh.at[col_v.at[j]], add=True)

        plsc.subcore_barrier()
        pltpu.async_copy(
            acc_sh.at[pl.ds(s * RPT, RPT)],
            out_hbm.at[c, pl.ds(s * RPT, RPT)],
            sem,
        ).wait()

    return k(col3, ones, zeros)


def _scatter_partials(hlo, hhi, row3, col3, zeros):
    @functools.partial(
        pl.kernel,
        mesh=_mesh,
        compiler_params=_params,
        out_type=jax.ShapeDtypeStruct((NC, NPAD, DH), jnp.float32),
        scratch_types=[
            pltpu.VMEM((NCH_SCAT, CHUNK), jnp.int32),
            pltpu.VMEM((NCH_SCAT, CHUNK), jnp.int32),
            pltpu.VMEM((CHUNK, DH), jnp.float32),
            pltpu.VMEM((CHUNK, DH), jnp.float32),
            pltpu.VMEM_SHARED((NPAD, DH), jnp.float32),
            pltpu.SemaphoreType.DMA,
            pltpu.SemaphoreType.DMA,
            pltpu.SemaphoreType.DMA,
            pltpu.SemaphoreType.DMA,
            pltpu.SemaphoreType.DMA,
        ],
    )
    def k(hlo_hbm, hhi_hbm, row_hbm, col_hbm, zeros_hbm, out_hbm,
          row_v, col_v, buf0, buf1, acc_sh, g0, g1, s0, s1, sem2):
        c = lax.axis_index("c")
        s = lax.axis_index("s")
        pltpu.async_copy(zeros_hbm, acc_sh.at[pl.ds(s * RPT, RPT)], sem2).wait()
        pltpu.async_copy(row_hbm.at[s], row_v, g0).wait()
        pltpu.async_copy(col_hbm.at[s], col_v, g1).wait()
        plsc.subcore_barrier()

        def run(h_hbm):
            pltpu.async_copy(h_hbm.at[row_v.at[0]], buf0, g0)
            pltpu.async_copy(h_hbm.at[row_v.at[1]], buf1, g1)

            @pl.loop(0, NCH_SCAT, step=2)
            def _(j):
                pltpu.make_async_copy(h_hbm.at[row_v.at[0]], buf0, g0).wait()
                pltpu.async_copy(buf0, acc_sh.at[col_v.at[j]], s0, add=True)
                pltpu.make_async_copy(h_hbm.at[row_v.at[0]], buf1, g1).wait()
                pltpu.async_copy(buf1, acc_sh.at[col_v.at[j + 1]], s1, add=True)

                @pl.when(j + 2 < NCH_SCAT)
                def _():
                    pltpu.make_async_copy(buf0, acc_sh.at[col_v.at[0]], s0).wait()
                    pltpu.async_copy(h_hbm.at[row_v.at[j + 2]], buf0, g0)

                @pl.when(j + 3 < NCH_SCAT)
                def _():
                    pltpu.make_async_copy(buf1, acc_sh.at[col_v.at[0]], s1).wait()
                    pltpu.async_copy(h_hbm.at[row_v.at[j + 3]], buf1, g1)

            pltpu.make_async_copy(buf0, acc_sh.at[col_v.at[0]], s0).wait()
            pltpu.make_async_copy(buf1, acc_sh.at[col_v.at[0]], s1).wait()

        @pl.when(c == 0)
        def _():
            run(hlo_hbm)

        @pl.when(c == 1)
        def _():
            run(hhi_hbm)

        plsc.subcore_barrier()
        pltpu.async_copy(
            acc_sh.at[pl.ds(s * RPT, RPT)],
            out_hbm.at[c, pl.ds(s * RPT, RPT)],
            sem2,
        ).wait()

    return k(hlo, hhi, row3, col3, zeros)


_R = 1000


def _dis_block(dg_ref):
    d16 = dg_ref[0] + dg_ref[1] + 1.0
    return lax.rsqrt(d16)[:, :1]


def _tc_first(x, W1, degp):
    def body(x_ref, w_ref, dg_ref, lo_ref, hi_ref):
        dis = _dis_block(dg_ref)
        h = jnp.dot(x_ref[...], w_ref[...],
                    preferred_element_type=jnp.float32) * dis
        lo_ref[...] = h[:, :DH]
        hi_ref[...] = h[:, DH:]

    return pl.pallas_call(
        body,
        grid=(N_NODES // _R,),
        in_specs=[
            pl.BlockSpec((_R, D), lambda i: (i, 0)),
            pl.BlockSpec((D, D), lambda i: (0, 0)),
            pl.BlockSpec((NC, _R, 16), lambda i: (0, i, 0)),
        ],
        out_specs=[
            pl.BlockSpec((_R, DH), lambda i: (i, 0)),
            pl.BlockSpec((_R, DH), lambda i: (i, 0)),
        ],
        out_shape=[
            jax.ShapeDtypeStruct((N_NODES, DH), jnp.float32),
            jax.ShapeDtypeStruct((N_NODES, DH), jnp.float32),
        ],
    )(x, W1, degp)


def _tc_mid(Sp, hlo, hhi, degp, b1, W2):
    def body(sp_ref, lo_ref, hi_ref, dg_ref, b_ref, w_ref,
             x1_ref, h2lo_ref, h2hi_ref):
        dis = _dis_block(dg_ref)
        h1 = jnp.concatenate([lo_ref[...], hi_ref[...]], axis=1)
        agg = jnp.concatenate([sp_ref[0], sp_ref[1]], axis=1) + h1
        x1 = jnp.maximum(agg * dis + b_ref[...], 0.0)
        x1_ref[...] = x1
        h2 = jnp.dot(x1, w_ref[...],
                     preferred_element_type=jnp.float32) * dis
        h2lo_ref[...] = h2[:, :DH]
        h2hi_ref[...] = h2[:, DH:]

    return pl.pallas_call(
        body,
        grid=(N_NODES // _R,),
        in_specs=[
            pl.BlockSpec((NC, _R, DH), lambda i: (0, i, 0)),
            pl.BlockSpec((_R, DH), lambda i: (i, 0)),
            pl.BlockSpec((_R, DH), lambda i: (i, 0)),
            pl.BlockSpec((NC, _R, 16), lambda i: (0, i, 0)),
            pl.BlockSpec((1, D), lambda i: (0, 0)),
            pl.BlockSpec((D, D), lambda i: (0, 0)),
        ],
        out_specs=[
            pl.BlockSpec((_R, D), lambda i: (i, 0)),
            pl.BlockSpec((_R, DH), lambda i: (i, 0)),
            pl.BlockSpec((_R, DH), lambda i: (i, 0)),
        ],
        out_shape=[
            jax.ShapeDtypeStruct((N_NODES, D), jnp.float32),
            jax.ShapeDtypeStruct((N_NODES, DH), jnp.float32),
            jax.ShapeDtypeStruct((N_NODES, DH), jnp.float32),
        ],
    )(Sp, hlo, hhi, degp, b1, W2)


def _tc_last(Sp, h2lo, h2hi, degp, b2, x1, Wlin, blin):
    def body(sp_ref, lo_ref, hi_ref, dg_ref, b_ref, x1_ref, wl_ref, bl_ref,
             o_ref):
        dis = _dis_block(dg_ref)
        h2 = jnp.concatenate([lo_ref[...], hi_ref[...]], axis=1)
        agg = jnp.concatenate([sp_ref[0], sp_ref[1]], axis=1) + h2
        x2 = jnp.maximum(agg * dis + b_ref[...], 0.0)
        hsum = x1_ref[...] + x2
        logits = jnp.dot(
            hsum, wl_ref[...], preferred_element_type=jnp.float32) + bl_ref[...]
        m = jnp.max(logits, axis=1, keepdims=True)
        lse = jnp.log(jnp.sum(jnp.exp(logits - m), axis=1, keepdims=True))
        o_ref[...] = logits - m - lse

    return pl.pallas_call(
        body,
        grid=(N_NODES // _R,),
        in_specs=[
            pl.BlockSpec((NC, _R, DH), lambda i: (0, i, 0)),
            pl.BlockSpec((_R, DH), lambda i: (i, 0)),
            pl.BlockSpec((_R, DH), lambda i: (i, 0)),
            pl.BlockSpec((NC, _R, 16), lambda i: (0, i, 0)),
            pl.BlockSpec((1, D), lambda i: (0, 0)),
            pl.BlockSpec((_R, D), lambda i: (i, 0)),
            pl.BlockSpec((D, N_CLASSES), lambda i: (0, 0)),
            pl.BlockSpec((1, N_CLASSES), lambda i: (0, 0)),
        ],
        out_specs=pl.BlockSpec((_R, N_CLASSES), lambda i: (i, 0)),
        out_shape=jax.ShapeDtypeStruct((N_NODES, N_CLASSES), jnp.float32),
    )(Sp, h2lo, h2hi, degp, b2, x1, Wlin, blin)


def kernel(x, edge_index, W1, b1, W2, b2, Wlin, blin):
    ei = edge_index.astype(jnp.int32)
    n_extra = E_PAD - N_EDGES
    pad_rows = (jnp.arange(n_extra, dtype=jnp.int32) * 131) % N_NODES
    pad_cols = PAD_COL + (jnp.arange(n_extra, dtype=jnp.int32) % (NPAD - PAD_COL))
    row_p = jnp.concatenate([ei[0], pad_rows])
    col_p = jnp.concatenate([ei[1], pad_cols])
    col_deg = col_p.reshape(NW, NCH_DEG, CHUNK)
    row3 = row_p.reshape(NS, NCH_SCAT, CHUNK)
    col3 = col_p.reshape(NS, NCH_SCAT, CHUNK)

    ones16 = jnp.ones((CHUNK, 16), jnp.float32)
    zeros16 = jnp.zeros((RPT, 16), jnp.float32)
    zerosH = jnp.zeros((RPT, DH), jnp.float32)

    degp = _deg_partials(col_deg, ones16, zeros16)
    h1lo, h1hi = _tc_first(x, W1, degp)
    S1 = _scatter_partials(h1lo, h1hi, row3, col3, zerosH)
    x1, h2lo, h2hi = _tc_mid(S1, h1lo, h1hi, degp, b1.reshape(1, D), W2)
    S2 = _scatter_partials(h2lo, h2hi, row3, col3, zerosH)
    return _tc_last(S2, h2lo, h2hi, degp, b2.reshape(1, D), x1,
                    Wlin, blin.reshape(1, N_CLASSES))


# speedup vs baseline: 1.1056x; 1.1056x over previous
import functools

import jax
import jax.numpy as jnp
from jax import lax
from jax.experimental import pallas as pl
from jax.experimental.pallas import tpu as pltpu
from jax.experimental.pallas import tpu_sc as plsc

N_NODES = 10000
N_EDGES = 320000
D = 128
DH = D // 2
N_CLASSES = 40

NC = 2
NS = 16
NW = NC * NS
CHUNK = 256
E_PAD = 327680
PAD_COL = 10200
NCH_DEG = E_PAD // (NW * CHUNK)
NCH_SCAT = E_PAD // (NS * CHUNK)
NPAD = 10240
RPT = NPAD // NS

_mesh = plsc.VectorSubcoreMesh(core_axis_name="c", subcore_axis_name="s")
_params = pltpu.CompilerParams(use_tc_tiling_on_sc=False)


def _deg_partials(col3, ones, zeros):
    @functools.partial(
        pl.kernel,
        mesh=_mesh,
        compiler_params=_params,
        out_type=jax.ShapeDtypeStruct((NC, NPAD, 16), jnp.float32),
        scratch_types=[
            pltpu.VMEM((NCH_DEG, CHUNK), jnp.int32),
            pltpu.VMEM((CHUNK, 16), jnp.float32),
            pltpu.VMEM_SHARED((NPAD, 16), jnp.float32),
            pltpu.SemaphoreType.DMA,
        ],
    )
    def k(col_hbm, ones_hbm, zeros_hbm, out_hbm, col_v, ones_v, acc_sh, sem):
        c = lax.axis_index("c")
        s = lax.axis_index("s")
        wid = c * NS + s
        pltpu.async_copy(zeros_hbm, acc_sh.at[pl.ds(s * RPT, RPT)], sem).wait()
        pltpu.async_copy(ones_hbm, ones_v, sem).wait()
        pltpu.async_copy(col_hbm.at[wid], col_v, sem).wait()
        plsc.subcore_barrier()

        @pl.loop(0, NCH_DEG)
        def _(j):
            pltpu.sync_copy(ones_v, acc_sh.at[col_v.at[j]], add=True)

        plsc.subcore_barrier()
        pltpu.async_copy(
            acc_sh.at[pl.ds(s * RPT, RPT)],
            out_hbm.at[c, pl.ds(s * RPT, RPT)],
            sem,
        ).wait()

    return k(col3, ones, zeros)


def _scatter_partials(hlo, hhi, row3, col3, zeros):
    @functools.partial(
        pl.kernel,
        mesh=_mesh,
        compiler_params=_params,
        out_type=jax.ShapeDtypeStruct((NC, NPAD, DH), jnp.float32),
        scratch_types=[
            pltpu.VMEM((NCH_SCAT, CHUNK), jnp.int32),
            pltpu.VMEM((NCH_SCAT, CHUNK), jnp.int32),
            pltpu.VMEM((CHUNK, DH), jnp.float32),
            pltpu.VMEM((CHUNK, DH), jnp.float32),
            pltpu.VMEM_SHARED((NPAD, DH), jnp.float32),
            pltpu.SemaphoreType.DMA,
            pltpu.SemaphoreType.DMA,
            pltpu.SemaphoreType.DMA,
            pltpu.SemaphoreType.DMA,
            pltpu.SemaphoreType.DMA,
        ],
    )
    def k(hlo_hbm, hhi_hbm, row_hbm, col_hbm, zeros_hbm, out_hbm,
          row_v, col_v, buf0, buf1, acc_sh, g0, g1, s0, s1, sem2):
        c = lax.axis_index("c")
        s = lax.axis_index("s")
        pltpu.async_copy(zeros_hbm, acc_sh.at[pl.ds(s * RPT, RPT)], sem2).wait()
        pltpu.async_copy(row_hbm.at[s], row_v, g0).wait()
        pltpu.async_copy(col_hbm.at[s], col_v, g1).wait()
        plsc.subcore_barrier()

        def run(h_hbm):
            pltpu.async_copy(h_hbm.at[row_v.at[0]], buf0, g0)

            @pl.loop(0, NCH_SCAT, step=2)
            def _(j):
                pltpu.make_async_copy(h_hbm.at[row_v.at[0]], buf0, g0).wait()
                pltpu.async_copy(h_hbm.at[row_v.at[j + 1]], buf1, g1)
                pltpu.sync_copy(buf0, acc_sh.at[col_v.at[j]], add=True)
                pltpu.make_async_copy(h_hbm.at[row_v.at[0]], buf1, g1).wait()

                @pl.when(j + 2 < NCH_SCAT)
                def _():
                    pltpu.async_copy(h_hbm.at[row_v.at[j + 2]], buf0, g0)

                pltpu.sync_copy(buf1, acc_sh.at[col_v.at[j + 1]], add=True)

        @pl.when(c == 0)
        def _():
            run(hlo_hbm)

        @pl.when(c == 1)
        def _():
            run(hhi_hbm)

        plsc.subcore_barrier()
        pltpu.async_copy(
            acc_sh.at[pl.ds(s * RPT, RPT)],
            out_hbm.at[c, pl.ds(s * RPT, RPT)],
            sem2,
        ).wait()

    return k(hlo, hhi, row3, col3, zeros)


_R = 1000


def _dis_block(dg_ref):
    d16 = dg_ref[0] + dg_ref[1] + 1.0
    return lax.rsqrt(d16)[:, :1]


def _tc_first(x, W1, degp):
    def body(x_ref, w_ref, dg_ref, lo_ref, hi_ref):
        dis = _dis_block(dg_ref)
        h = jnp.dot(x_ref[...], w_ref[...],
                    preferred_element_type=jnp.float32) * dis
        lo_ref[...] = h[:, :DH]
        hi_ref[...] = h[:, DH:]

    return pl.pallas_call(
        body,
        grid=(N_NODES // _R,),
        in_specs=[
            pl.BlockSpec((_R, D), lambda i: (i, 0)),
            pl.BlockSpec((D, D), lambda i: (0, 0)),
            pl.BlockSpec((NC, _R, 16), lambda i: (0, i, 0)),
        ],
        out_specs=[
            pl.BlockSpec((_R, DH), lambda i: (i, 0)),
            pl.BlockSpec((_R, DH), lambda i: (i, 0)),
        ],
        out_shape=[
            jax.ShapeDtypeStruct((N_NODES, DH), jnp.float32),
            jax.ShapeDtypeStruct((N_NODES, DH), jnp.float32),
        ],
    )(x, W1, degp)


def _tc_mid(Sp, hlo, hhi, degp, b1, W2):
    def body(sp_ref, lo_ref, hi_ref, dg_ref, b_ref, w_ref,
             x1_ref, h2lo_ref, h2hi_ref):
        dis = _dis_block(dg_ref)
        h1 = jnp.concatenate([lo_ref[...], hi_ref[...]], axis=1)
        agg = jnp.concatenate([sp_ref[0], sp_ref[1]], axis=1) + h1
        x1 = jnp.maximum(agg * dis + b_ref[...], 0.0)
        x1_ref[...] = x1
        h2 = jnp.dot(x1, w_ref[...],
                     preferred_element_type=jnp.float32) * dis
        h2lo_ref[...] = h2[:, :DH]
        h2hi_ref[...] = h2[:, DH:]

    return pl.pallas_call(
        body,
        grid=(N_NODES // _R,),
        in_specs=[
            pl.BlockSpec((NC, _R, DH), lambda i: (0, i, 0)),
            pl.BlockSpec((_R, DH), lambda i: (i, 0)),
            pl.BlockSpec((_R, DH), lambda i: (i, 0)),
            pl.BlockSpec((NC, _R, 16), lambda i: (0, i, 0)),
            pl.BlockSpec((1, D), lambda i: (0, 0)),
            pl.BlockSpec((D, D), lambda i: (0, 0)),
        ],
        out_specs=[
            pl.BlockSpec((_R, D), lambda i: (i, 0)),
            pl.BlockSpec((_R, DH), lambda i: (i, 0)),
            pl.BlockSpec((_R, DH), lambda i: (i, 0)),
        ],
        out_shape=[
            jax.ShapeDtypeStruct((N_NODES, D), jnp.float32),
            jax.ShapeDtypeStruct((N_NODES, DH), jnp.float32),
            jax.ShapeDtypeStruct((N_NODES, DH), jnp.float32),
        ],
    )(Sp, hlo, hhi, degp, b1, W2)


def _tc_last(Sp, h2lo, h2hi, degp, b2, x1, Wlin, blin):
    def body(sp_ref, lo_ref, hi_ref, dg_ref, b_ref, x1_ref, wl_ref, bl_ref,
             o_ref):
        dis = _dis_block(dg_ref)
        h2 = jnp.concatenate([lo_ref[...], hi_ref[...]], axis=1)
        agg = jnp.concatenate([sp_ref[0], sp_ref[1]], axis=1) + h2
        x2 = jnp.maximum(agg * dis + b_ref[...], 0.0)
        hsum = x1_ref[...] + x2
        logits = jnp.dot(
            hsum, wl_ref[...], preferred_element_type=jnp.float32) + bl_ref[...]
        m = jnp.max(logits, axis=1, keepdims=True)
        lse = jnp.log(jnp.sum(jnp.exp(logits - m), axis=1, keepdims=True))
        o_ref[...] = logits - m - lse

    return pl.pallas_call(
        body,
        grid=(N_NODES // _R,),
        in_specs=[
            pl.BlockSpec((NC, _R, DH), lambda i: (0, i, 0)),
            pl.BlockSpec((_R, DH), lambda i: (i, 0)),
            pl.BlockSpec((_R, DH), lambda i: (i, 0)),
            pl.BlockSpec((NC, _R, 16), lambda i: (0, i, 0)),
            pl.BlockSpec((1, D), lambda i: (0, 0)),
            pl.BlockSpec((_R, D), lambda i: (i, 0)),
            pl.BlockSpec((D, N_CLASSES), lambda i: (0, 0)),
            pl.BlockSpec((1, N_CLASSES), lambda i: (0, 0)),
        ],
        out_specs=pl.BlockSpec((_R, N_CLASSES), lambda i: (i, 0)),
        out_shape=jax.ShapeDtypeStruct((N_NODES, N_CLASSES), jnp.float32),
    )(Sp, h2lo, h2hi, degp, b2, x1, Wlin, blin)


def kernel(x, edge_index, W1, b1, W2, b2, Wlin, blin):
    ei = edge_index.astype(jnp.int32)
    n_extra = E_PAD - N_EDGES
    pad_rows = (jnp.arange(n_extra, dtype=jnp.int32) * 131) % N_NODES
    pad_cols = PAD_COL + (jnp.arange(n_extra, dtype=jnp.int32) % (NPAD - PAD_COL))
    row_p = jnp.concatenate([ei[0], pad_rows])
    col_p = jnp.concatenate([ei[1], pad_cols])
    col_deg = col_p.reshape(NW, NCH_DEG, CHUNK)
    row3 = row_p.reshape(NS, NCH_SCAT, CHUNK)
    col3 = col_p.reshape(NS, NCH_SCAT, CHUNK)

    ones16 = jnp.ones((CHUNK, 16), jnp.float32)
    zeros16 = jnp.zeros((RPT, 16), jnp.float32)
    zerosH = jnp.zeros((RPT, DH), jnp.float32)

    degp = _deg_partials(col_deg, ones16, zeros16)
    h1lo, h1hi = _tc_first(x, W1, degp)
    S1 = _scatter_partials(h1lo, h1hi, row3, col3, zerosH)
    x1, h2lo, h2hi = _tc_mid(S1, h1lo, h1hi, degp, b1.reshape(1, D), W2)
    S2 = _scatter_partials(h2lo, h2hi, row3, col3, zerosH)
    return _tc_last(S2, h2lo, h2hi, degp, b2.reshape(1, D), x1,
                    Wlin, blin.reshape(1, N_CLASSES))


# speedup vs baseline: 1.1289x; 1.0211x over previous
import functools

import jax
import jax.numpy as jnp
from jax import lax
from jax.experimental import pallas as pl
from jax.experimental.pallas import tpu as pltpu
from jax.experimental.pallas import tpu_sc as plsc

N_NODES = 10000
N_EDGES = 320000
D = 128
DH = D // 2
N_CLASSES = 40

NC = 2
NS = 16
NW = NC * NS
CHUNK = 256
E_PAD = 327680
PAD_COL = 10200
NCH_DEG = E_PAD // (NW * CHUNK)
NCH_SCAT = E_PAD // (NS * CHUNK)
NPAD = 10240
RPT = NPAD // NS

_mesh = plsc.VectorSubcoreMesh(core_axis_name="c", subcore_axis_name="s")
_params = pltpu.CompilerParams(use_tc_tiling_on_sc=False)


def _deg_partials(col3, ones, zeros):
    @functools.partial(
        pl.kernel,
        mesh=_mesh,
        compiler_params=_params,
        out_type=jax.ShapeDtypeStruct((NC, NPAD, 16), jnp.float32),
        scratch_types=[
            pltpu.VMEM((NCH_DEG, CHUNK), jnp.int32),
            pltpu.VMEM((CHUNK, 16), jnp.float32),
            pltpu.VMEM_SHARED((NPAD, 16), jnp.float32),
            pltpu.SemaphoreType.DMA,
        ],
    )
    def k(col_hbm, ones_hbm, zeros_hbm, out_hbm, col_v, ones_v, acc_sh, sem):
        c = lax.axis_index("c")
        s = lax.axis_index("s")
        wid = c * NS + s
        pltpu.async_copy(zeros_hbm, acc_sh.at[pl.ds(s * RPT, RPT)], sem).wait()
        pltpu.async_copy(ones_hbm, ones_v, sem).wait()
        pltpu.async_copy(col_hbm.at[wid], col_v, sem).wait()
        plsc.subcore_barrier()

        @pl.loop(0, NCH_DEG)
        def _(j):
            pltpu.sync_copy(ones_v, acc_sh.at[col_v.at[j]], add=True)

        plsc.subcore_barrier()
        pltpu.async_copy(
            acc_sh.at[pl.ds(s * RPT, RPT)],
            out_hbm.at[c, pl.ds(s * RPT, RPT)],
            sem,
        ).wait()

    return k(col3, ones, zeros)


def _scatter_partials(hlo, hhi, row3, col3, zeros):
    @functools.partial(
        pl.kernel,
        mesh=_mesh,
        compiler_params=_params,
        out_type=jax.ShapeDtypeStruct((NC, NPAD, DH), jnp.float32),
        scratch_types=[
            pltpu.VMEM((NCH_SCAT, CHUNK), jnp.int32),
            pltpu.VMEM((NCH_SCAT, CHUNK), jnp.int32),
            pltpu.VMEM((CHUNK, DH), jnp.float32),
            pltpu.VMEM((CHUNK, DH), jnp.float32),
            pltpu.VMEM_SHARED((NPAD, DH), jnp.float32),
            pltpu.SemaphoreType.DMA,
            pltpu.SemaphoreType.DMA,
            pltpu.SemaphoreType.DMA,
            pltpu.SemaphoreType.DMA,
            pltpu.SemaphoreType.DMA,
        ],
    )
    def k(hlo_hbm, hhi_hbm, row_hbm, col_hbm, zeros_hbm, out_hbm,
          row_v, col_v, buf0, buf1, acc_sh, g0, g1, s0, s1, sem2):
        c = lax.axis_index("c")
        s = lax.axis_index("s")
        pltpu.async_copy(zeros_hbm, acc_sh.at[pl.ds(s * RPT, RPT)], sem2).wait()
        pltpu.async_copy(row_hbm.at[s], row_v, g0).wait()
        pltpu.async_copy(col_hbm.at[s], col_v, g1).wait()
        plsc.subcore_barrier()

        def run(h_hbm):
            pltpu.async_copy(h_hbm.at[row_v.at[0]], buf0, g0)

            @pl.loop(0, NCH_SCAT, step=2)
            def _(j):
                pltpu.make_async_copy(h_hbm.at[row_v.at[0]], buf0, g0).wait()
                pltpu.async_copy(h_hbm.at[row_v.at[j + 1]], buf1, g1)
                pltpu.sync_copy(buf0, acc_sh.at[col_v.at[j]], add=True)
                pltpu.make_async_copy(h_hbm.at[row_v.at[0]], buf1, g1).wait()

                @pl.when(j + 2 < NCH_SCAT)
                def _():
                    pltpu.async_copy(h_hbm.at[row_v.at[j + 2]], buf0, g0)

                pltpu.sync_copy(buf1, acc_sh.at[col_v.at[j + 1]], add=True)

        @pl.when(c == 0)
        def _():
            run(hlo_hbm)

        @pl.when(c == 1)
        def _():
            run(hhi_hbm)

        plsc.subcore_barrier()
        pltpu.async_copy(
            acc_sh.at[pl.ds(s * RPT, RPT)],
            out_hbm.at[c, pl.ds(s * RPT, RPT)],
            sem2,
        ).wait()

    return k(hlo, hhi, row3, col3, zeros)


_R = 5000


def _dis_block(dg_ref):
    d16 = dg_ref[0] + dg_ref[1] + 1.0
    return lax.rsqrt(d16)[:, :1]


def _tc_first(x, W1, degp):
    def body(x_ref, w_ref, dg_ref, lo_ref, hi_ref):
        dis = _dis_block(dg_ref)
        h = jnp.dot(x_ref[...], w_ref[...],
                    preferred_element_type=jnp.float32) * dis
        lo_ref[...] = h[:, :DH]
        hi_ref[...] = h[:, DH:]

    return pl.pallas_call(
        body,
        grid=(N_NODES // _R,),
        in_specs=[
            pl.BlockSpec((_R, D), lambda i: (i, 0)),
            pl.BlockSpec((D, D), lambda i: (0, 0)),
            pl.BlockSpec((NC, _R, 16), lambda i: (0, i, 0)),
        ],
        out_specs=[
            pl.BlockSpec((_R, DH), lambda i: (i, 0)),
            pl.BlockSpec((_R, DH), lambda i: (i, 0)),
        ],
        out_shape=[
            jax.ShapeDtypeStruct((N_NODES, DH), jnp.float32),
            jax.ShapeDtypeStruct((N_NODES, DH), jnp.float32),
        ],
    )(x, W1, degp)


def _tc_mid(Sp, hlo, hhi, degp, b1, W2):
    def body(sp_ref, lo_ref, hi_ref, dg_ref, b_ref, w_ref,
             x1_ref, h2lo_ref, h2hi_ref):
        dis = _dis_block(dg_ref)
        h1 = jnp.concatenate([lo_ref[...], hi_ref[...]], axis=1)
        agg = jnp.concatenate([sp_ref[0], sp_ref[1]], axis=1) + h1
        x1 = jnp.maximum(agg * dis + b_ref[...], 0.0)
        x1_ref[...] = x1
        h2 = jnp.dot(x1, w_ref[...],
                     preferred_element_type=jnp.float32) * dis
        h2lo_ref[...] = h2[:, :DH]
        h2hi_ref[...] = h2[:, DH:]

    return pl.pallas_call(
        body,
        grid=(N_NODES // _R,),
        in_specs=[
            pl.BlockSpec((NC, _R, DH), lambda i: (0, i, 0)),
            pl.BlockSpec((_R, DH), lambda i: (i, 0)),
            pl.BlockSpec((_R, DH), lambda i: (i, 0)),
            pl.BlockSpec((NC, _R, 16), lambda i: (0, i, 0)),
            pl.BlockSpec((1, D), lambda i: (0, 0)),
            pl.BlockSpec((D, D), lambda i: (0, 0)),
        ],
        out_specs=[
            pl.BlockSpec((_R, D), lambda i: (i, 0)),
            pl.BlockSpec((_R, DH), lambda i: (i, 0)),
            pl.BlockSpec((_R, DH), lambda i: (i, 0)),
        ],
        out_shape=[
            jax.ShapeDtypeStruct((N_NODES, D), jnp.float32),
            jax.ShapeDtypeStruct((N_NODES, DH), jnp.float32),
            jax.ShapeDtypeStruct((N_NODES, DH), jnp.float32),
        ],
    )(Sp, hlo, hhi, degp, b1, W2)


def _tc_last(Sp, h2lo, h2hi, degp, b2, x1, Wlin, blin):
    def body(sp_ref, lo_ref, hi_ref, dg_ref, b_ref, x1_ref, wl_ref, bl_ref,
             o_ref):
        dis = _dis_block(dg_ref)
        h2 = jnp.concatenate([lo_ref[...], hi_ref[...]], axis=1)
        agg = jnp.concatenate([sp_ref[0], sp_ref[1]], axis=1) + h2
        x2 = jnp.maximum(agg * dis + b_ref[...], 0.0)
        hsum = x1_ref[...] + x2
        logits = jnp.dot(
            hsum, wl_ref[...], preferred_element_type=jnp.float32) + bl_ref[...]
        m = jnp.max(logits, axis=1, keepdims=True)
        lse = jnp.log(jnp.sum(jnp.exp(logits - m), axis=1, keepdims=True))
        o_ref[...] = logits - m - lse

    return pl.pallas_call(
        body,
        grid=(N_NODES // _R,),
        in_specs=[
            pl.BlockSpec((NC, _R, DH), lambda i: (0, i, 0)),
            pl.BlockSpec((_R, DH), lambda i: (i, 0)),
            pl.BlockSpec((_R, DH), lambda i: (i, 0)),
            pl.BlockSpec((NC, _R, 16), lambda i: (0, i, 0)),
            pl.BlockSpec((1, D), lambda i: (0, 0)),
            pl.BlockSpec((_R, D), lambda i: (i, 0)),
            pl.BlockSpec((D, N_CLASSES), lambda i: (0, 0)),
            pl.BlockSpec((1, N_CLASSES), lambda i: (0, 0)),
        ],
        out_specs=pl.BlockSpec((_R, N_CLASSES), lambda i: (i, 0)),
        out_shape=jax.ShapeDtypeStruct((N_NODES, N_CLASSES), jnp.float32),
    )(Sp, h2lo, h2hi, degp, b2, x1, Wlin, blin)


def kernel(x, edge_index, W1, b1, W2, b2, Wlin, blin):
    ei = edge_index.astype(jnp.int32)
    n_extra = E_PAD - N_EDGES
    pad_rows = (jnp.arange(n_extra, dtype=jnp.int32) * 131) % N_NODES
    pad_cols = PAD_COL + (jnp.arange(n_extra, dtype=jnp.int32) % (NPAD - PAD_COL))
    row_p = jnp.concatenate([ei[0], pad_rows])
    col_p = jnp.concatenate([ei[1], pad_cols])
    col_deg = col_p.reshape(NW, NCH_DEG, CHUNK)
    row3 = row_p.reshape(NS, NCH_SCAT, CHUNK)
    col3 = col_p.reshape(NS, NCH_SCAT, CHUNK)

    ones16 = jnp.ones((CHUNK, 16), jnp.float32)
    zeros16 = jnp.zeros((RPT, 16), jnp.float32)
    zerosH = jnp.zeros((RPT, DH), jnp.float32)

    degp = _deg_partials(col_deg, ones16, zeros16)
    h1lo, h1hi = _tc_first(x, W1, degp)
    S1 = _scatter_partials(h1lo, h1hi, row3, col3, zerosH)
    x1, h2lo, h2hi = _tc_mid(S1, h1lo, h1hi, degp, b1.reshape(1, D), W2)
    S2 = _scatter_partials(h2lo, h2hi, row3, col3, zerosH)
    return _tc_last(S2, h2lo, h2hi, degp, b2.reshape(1, D), x1,
                    Wlin, blin.reshape(1, N_CLASSES))


# speedup vs baseline: 1.2783x; 1.1324x over previous
import functools

import jax
import jax.numpy as jnp
from jax import lax
from jax.experimental import pallas as pl
from jax.experimental.pallas import tpu as pltpu
from jax.experimental.pallas import tpu_sc as plsc

N_NODES = 10000
N_EDGES = 320000
D = 128
DH = D // 2
N_CLASSES = 40

NC = 2
NS = 16
NW = NC * NS
CHUNK = 256
E_PAD = 327680
PAD_COL = 10200
NCH_DEG = E_PAD // (NW * CHUNK)
NCH_SCAT = E_PAD // (NS * CHUNK)
NPAD = 10240
RPT = NPAD // NS

_mesh = plsc.VectorSubcoreMesh(core_axis_name="c", subcore_axis_name="s")
_params = pltpu.CompilerParams(use_tc_tiling_on_sc=False)


def _deg_partials(col3, ones, zeros):
    @functools.partial(
        pl.kernel,
        mesh=_mesh,
        compiler_params=_params,
        out_type=jax.ShapeDtypeStruct((NPAD, 128), jnp.float32),
        scratch_types=[
            pltpu.VMEM((NCH_DEG, CHUNK), jnp.int32),
            pltpu.VMEM((CHUNK, 16), jnp.float32),
            pltpu.VMEM_SHARED((NPAD, 16), jnp.float32),
            pltpu.SemaphoreType.DMA,
        ],
    )
    def k(col_hbm, ones_hbm, zeros_hbm, out_hbm, col_v, ones_v, acc_sh, sem):
        c = lax.axis_index("c")
        s = lax.axis_index("s")
        wid = c * NS + s
        pltpu.async_copy(zeros_hbm, acc_sh.at[pl.ds(s * RPT, RPT)], sem).wait()
        pltpu.async_copy(ones_hbm, ones_v, sem).wait()
        pltpu.async_copy(col_hbm.at[wid], col_v, sem).wait()
        plsc.subcore_barrier()

        @pl.loop(0, NCH_DEG)
        def _(j):
            pltpu.sync_copy(ones_v, acc_sh.at[col_v.at[j]], add=True)

        plsc.subcore_barrier()
        pltpu.async_copy(
            acc_sh.at[pl.ds(s * RPT, RPT)],
            out_hbm.at[pl.ds(s * RPT, RPT), pl.ds(c * 16, 16)],
            sem,
        ).wait()

    return k(col3, ones, zeros)


def _scatter_partials(hview, rowlo3, rowhi3, col3, zeros):
    @functools.partial(
        pl.kernel,
        mesh=_mesh,
        compiler_params=_params,
        out_type=jax.ShapeDtypeStruct((NPAD, D), jnp.float32),
        scratch_types=[
            pltpu.VMEM((NCH_SCAT, CHUNK), jnp.int32),
            pltpu.VMEM((NCH_SCAT, CHUNK), jnp.int32),
            pltpu.VMEM((CHUNK, DH), jnp.float32),
            pltpu.VMEM((CHUNK, DH), jnp.float32),
            pltpu.VMEM_SHARED((NPAD, DH), jnp.float32),
            pltpu.SemaphoreType.DMA,
            pltpu.SemaphoreType.DMA,
            pltpu.SemaphoreType.DMA,
        ],
    )
    def k(h_hbm, rowlo_hbm, rowhi_hbm, col_hbm, zeros_hbm, out_hbm,
          row_v, col_v, buf0, buf1, acc_sh, g0, g1, sem2):
        c = lax.axis_index("c")
        s = lax.axis_index("s")
        pltpu.async_copy(zeros_hbm, acc_sh.at[pl.ds(s * RPT, RPT)], sem2).wait()
        pltpu.async_copy(col_hbm.at[s], col_v, g1).wait()

        @pl.when(c == 0)
        def _():
            pltpu.async_copy(rowlo_hbm.at[s], row_v, g0).wait()

        @pl.when(c == 1)
        def _():
            pltpu.async_copy(rowhi_hbm.at[s], row_v, g0).wait()

        plsc.subcore_barrier()

        pltpu.async_copy(h_hbm.at[row_v.at[0]], buf0, g0)

        @pl.loop(0, NCH_SCAT, step=2)
        def _(j):
            pltpu.make_async_copy(h_hbm.at[row_v.at[0]], buf0, g0).wait()
            pltpu.async_copy(h_hbm.at[row_v.at[j + 1]], buf1, g1)
            pltpu.sync_copy(buf0, acc_sh.at[col_v.at[j]], add=True)
            pltpu.make_async_copy(h_hbm.at[row_v.at[0]], buf1, g1).wait()

            @pl.when(j + 2 < NCH_SCAT)
            def _():
                pltpu.async_copy(h_hbm.at[row_v.at[j + 2]], buf0, g0)

            pltpu.sync_copy(buf1, acc_sh.at[col_v.at[j + 1]], add=True)

        plsc.subcore_barrier()
        pltpu.async_copy(
            acc_sh.at[pl.ds(s * RPT, RPT)],
            out_hbm.at[pl.ds(s * RPT, RPT), pl.ds(c * DH, DH)],
            sem2,
        ).wait()

    return k(hview, rowlo3, rowhi3, col3, zeros)


_R = N_NODES


def _dis_block(dgp_ref):
    d = dgp_ref[:N_NODES, 0:1] + dgp_ref[:N_NODES, 16:17] + 1.0
    return lax.rsqrt(d)


def _unpack(sp_ref, h_ref):
    return sp_ref[:N_NODES, :] + h_ref[...]


def _tc_first(x, W1, dgp):
    def body(x_ref, w_ref, dg_ref, o_ref):
        dis = _dis_block(dg_ref)
        h = jnp.dot(x_ref[...], w_ref[...],
                    preferred_element_type=jnp.float32) * dis
        o_ref[...] = h

    return pl.pallas_call(
        body,
        grid=(N_NODES // _R,),
        in_specs=[
            pl.BlockSpec((_R, D), lambda i: (i, 0)),
            pl.BlockSpec((D, D), lambda i: (0, 0)),
            pl.BlockSpec((NPAD, D), lambda i: (0, 0)),
        ],
        out_specs=pl.BlockSpec((_R, D), lambda i: (i, 0)),
        out_shape=jax.ShapeDtypeStruct((N_NODES, D), jnp.float32),
    )(x, W1, dgp)


def _tc_mid(Spv, h1, dgp, b1, W2):
    def body(sp_ref, h_ref, dg_ref, b_ref, w_ref, x1_ref, h2_ref):
        dis = _dis_block(dg_ref)
        agg = _unpack(sp_ref, h_ref)
        x1 = jnp.maximum(agg * dis + b_ref[...], 0.0)
        x1_ref[...] = x1
        h2_ref[...] = jnp.dot(x1, w_ref[...],
                              preferred_element_type=jnp.float32) * dis

    return pl.pallas_call(
        body,
        grid=(N_NODES // _R,),
        in_specs=[
            pl.BlockSpec((NPAD, D), lambda i: (0, 0)),
            pl.BlockSpec((_R, D), lambda i: (i, 0)),
            pl.BlockSpec((NPAD, D), lambda i: (0, 0)),
            pl.BlockSpec((1, D), lambda i: (0, 0)),
            pl.BlockSpec((D, D), lambda i: (0, 0)),
        ],
        out_specs=[
            pl.BlockSpec((_R, D), lambda i: (i, 0)),
            pl.BlockSpec((_R, D), lambda i: (i, 0)),
        ],
        out_shape=[
            jax.ShapeDtypeStruct((N_NODES, D), jnp.float32),
            jax.ShapeDtypeStruct((N_NODES, D), jnp.float32),
        ],
    )(Spv, h1, dgp, b1, W2)


def _tc_last(Spv, h2, dgp, b2, x1, Wlin, blin):
    def body(sp_ref, h_ref, dg_ref, b_ref, x1_ref, wl_ref, bl_ref, o_ref):
        dis = _dis_block(dg_ref)
        agg = _unpack(sp_ref, h_ref)
        x2 = jnp.maximum(agg * dis + b_ref[...], 0.0)
        hsum = x1_ref[...] + x2
        logits = jnp.dot(
            hsum, wl_ref[...], preferred_element_type=jnp.float32) + bl_ref[...]
        m = jnp.max(logits, axis=1, keepdims=True)
        lse = jnp.log(jnp.sum(jnp.exp(logits - m), axis=1, keepdims=True))
        o_ref[...] = logits - m - lse

    return pl.pallas_call(
        body,
        grid=(N_NODES // _R,),
        in_specs=[
            pl.BlockSpec((NPAD, D), lambda i: (0, 0)),
            pl.BlockSpec((_R, D), lambda i: (i, 0)),
            pl.BlockSpec((NPAD, D), lambda i: (0, 0)),
            pl.BlockSpec((1, D), lambda i: (0, 0)),
            pl.BlockSpec((_R, D), lambda i: (i, 0)),
            pl.BlockSpec((D, N_CLASSES), lambda i: (0, 0)),
            pl.BlockSpec((1, N_CLASSES), lambda i: (0, 0)),
        ],
        out_specs=pl.BlockSpec((_R, N_CLASSES), lambda i: (i, 0)),
        out_shape=jax.ShapeDtypeStruct((N_NODES, N_CLASSES), jnp.float32),
    )(Spv, h2, dgp, b2, x1, Wlin, blin)


def kernel(x, edge_index, W1, b1, W2, b2, Wlin, blin):
    ei = edge_index.astype(jnp.int32)
    n_extra = E_PAD - N_EDGES
    pad_rows = (jnp.arange(n_extra, dtype=jnp.int32) * 131) % N_NODES
    pad_cols = PAD_COL + (jnp.arange(n_extra, dtype=jnp.int32) % (NPAD - PAD_COL))
    row_p = jnp.concatenate([ei[0], pad_rows])
    col_p = jnp.concatenate([ei[1], pad_cols])
    col_deg = col_p.reshape(NW, NCH_DEG, CHUNK)
    rowlo3 = (2 * row_p).reshape(NS, NCH_SCAT, CHUNK)
    rowhi3 = (2 * row_p + 1).reshape(NS, NCH_SCAT, CHUNK)
    col3 = col_p.reshape(NS, NCH_SCAT, CHUNK)

    ones16 = jnp.ones((CHUNK, 16), jnp.float32)
    zeros16 = jnp.zeros((RPT, 16), jnp.float32)
    zerosH = jnp.zeros((RPT, DH), jnp.float32)

    dgp = _deg_partials(col_deg, ones16, zeros16)

    h1 = _tc_first(x, W1, dgp)
    S1 = _scatter_partials(h1.reshape(2 * N_NODES, DH), rowlo3, rowhi3, col3,
                           zerosH)
    x1, h2 = _tc_mid(S1, h1, dgp, b1.reshape(1, D), W2)
    S2 = _scatter_partials(h2.reshape(2 * N_NODES, DH), rowlo3, rowhi3, col3,
                           zerosH)
    return _tc_last(S2, h2, dgp, b2.reshape(1, D), x1,
                    Wlin, blin.reshape(1, N_CLASSES))


# speedup vs baseline: 1.3238x; 1.0355x over previous
import functools

import jax
import jax.numpy as jnp
from jax import lax
from jax.experimental import pallas as pl
from jax.experimental.pallas import tpu as pltpu
from jax.experimental.pallas import tpu_sc as plsc

N_NODES = 10000
N_EDGES = 320000
D = 128
DH = D // 2
N_CLASSES = 40

NC = 2
NS = 16
NW = NC * NS
CHUNK = 320
E_PAD = 327680
PAD_COL = 10200
NCH_DEG = E_PAD // (NW * CHUNK)
NCH_SCAT = E_PAD // (NS * CHUNK)
NPAD = 10240
RPT = NPAD // NS

_mesh = plsc.VectorSubcoreMesh(core_axis_name="c", subcore_axis_name="s")
_params = pltpu.CompilerParams(use_tc_tiling_on_sc=False)


def _deg_partials(col3, ones, zeros):
    @functools.partial(
        pl.kernel,
        mesh=_mesh,
        compiler_params=_params,
        out_type=jax.ShapeDtypeStruct((NPAD, 128), jnp.float32),
        scratch_types=[
            pltpu.VMEM((NCH_DEG, CHUNK), jnp.int32),
            pltpu.VMEM((CHUNK, 16), jnp.float32),
            pltpu.VMEM_SHARED((NPAD, 16), jnp.float32),
            pltpu.SemaphoreType.DMA,
        ],
    )
    def k(col_hbm, ones_hbm, zeros_hbm, out_hbm, col_v, ones_v, acc_sh, sem):
        c = lax.axis_index("c")
        s = lax.axis_index("s")
        wid = c * NS + s
        pltpu.async_copy(zeros_hbm, acc_sh.at[pl.ds(s * RPT, RPT)], sem).wait()
        pltpu.async_copy(ones_hbm, ones_v, sem).wait()
        pltpu.async_copy(col_hbm.at[wid], col_v, sem).wait()
        plsc.subcore_barrier()

        @pl.loop(0, NCH_DEG)
        def _(j):
            pltpu.sync_copy(ones_v, acc_sh.at[col_v.at[j]], add=True)

        plsc.subcore_barrier()
        pltpu.async_copy(
            acc_sh.at[pl.ds(s * RPT, RPT)],
            out_hbm.at[pl.ds(s * RPT, RPT), pl.ds(c * 16, 16)],
            sem,
        ).wait()

    return k(col3, ones, zeros)


def _scatter_partials(hview, rowlo3, rowhi3, col3, zeros):
    @functools.partial(
        pl.kernel,
        mesh=_mesh,
        compiler_params=_params,
        out_type=jax.ShapeDtypeStruct((NPAD, D), jnp.float32),
        scratch_types=[
            pltpu.VMEM((NCH_SCAT, CHUNK), jnp.int32),
            pltpu.VMEM((NCH_SCAT, CHUNK), jnp.int32),
            pltpu.VMEM((CHUNK, DH), jnp.float32),
            pltpu.VMEM((CHUNK, DH), jnp.float32),
            pltpu.VMEM_SHARED((NPAD, DH), jnp.float32),
            pltpu.SemaphoreType.DMA,
            pltpu.SemaphoreType.DMA,
            pltpu.SemaphoreType.DMA,
        ],
    )
    def k(h_hbm, rowlo_hbm, rowhi_hbm, col_hbm, zeros_hbm, out_hbm,
          row_v, col_v, buf0, buf1, acc_sh, g0, g1, sem2):
        c = lax.axis_index("c")
        s = lax.axis_index("s")
        pltpu.async_copy(zeros_hbm, acc_sh.at[pl.ds(s * RPT, RPT)], sem2).wait()
        pltpu.async_copy(col_hbm.at[s], col_v, g1).wait()

        @pl.when(c == 0)
        def _():
            pltpu.async_copy(rowlo_hbm.at[s], row_v, g0).wait()

        @pl.when(c == 1)
        def _():
            pltpu.async_copy(rowhi_hbm.at[s], row_v, g0).wait()

        plsc.subcore_barrier()

        pltpu.async_copy(h_hbm.at[row_v.at[0]], buf0, g0)

        @pl.loop(0, NCH_SCAT, step=2)
        def _(j):
            pltpu.make_async_copy(h_hbm.at[row_v.at[0]], buf0, g0).wait()
            pltpu.async_copy(h_hbm.at[row_v.at[j + 1]], buf1, g1)
            pltpu.sync_copy(buf0, acc_sh.at[col_v.at[j]], add=True)
            pltpu.make_async_copy(h_hbm.at[row_v.at[0]], buf1, g1).wait()

            @pl.when(j + 2 < NCH_SCAT)
            def _():
                pltpu.async_copy(h_hbm.at[row_v.at[j + 2]], buf0, g0)

            pltpu.sync_copy(buf1, acc_sh.at[col_v.at[j + 1]], add=True)

        plsc.subcore_barrier()
        pltpu.async_copy(
            acc_sh.at[pl.ds(s * RPT, RPT)],
            out_hbm.at[pl.ds(s * RPT, RPT), pl.ds(c * DH, DH)],
            sem2,
        ).wait()

    return k(hview, rowlo3, rowhi3, col3, zeros)


_R = N_NODES


def _dis_block(dgp_ref):
    d = dgp_ref[:N_NODES, 0:1] + dgp_ref[:N_NODES, 16:17] + 1.0
    return lax.rsqrt(d)


def _unpack(sp_ref, h_ref):
    return sp_ref[:N_NODES, :] + h_ref[...]


def _tc_first(x, W1, dgp):
    def body(x_ref, w_ref, dg_ref, o_ref):
        dis = _dis_block(dg_ref)
        h = jnp.dot(x_ref[...], w_ref[...],
                    preferred_element_type=jnp.float32) * dis
        o_ref[...] = h

    return pl.pallas_call(
        body,
        grid=(N_NODES // _R,),
        in_specs=[
            pl.BlockSpec((_R, D), lambda i: (i, 0)),
            pl.BlockSpec((D, D), lambda i: (0, 0)),
            pl.BlockSpec((NPAD, D), lambda i: (0, 0)),
        ],
        out_specs=pl.BlockSpec((_R, D), lambda i: (i, 0)),
        out_shape=jax.ShapeDtypeStruct((N_NODES, D), jnp.float32),
    )(x, W1, dgp)


def _tc_mid(Spv, h1, dgp, b1, W2):
    def body(sp_ref, h_ref, dg_ref, b_ref, w_ref, x1_ref, h2_ref):
        dis = _dis_block(dg_ref)
        agg = _unpack(sp_ref, h_ref)
        x1 = jnp.maximum(agg * dis + b_ref[...], 0.0)
        x1_ref[...] = x1
        h2_ref[...] = jnp.dot(x1, w_ref[...],
                              preferred_element_type=jnp.float32) * dis

    return pl.pallas_call(
        body,
        grid=(N_NODES // _R,),
        in_specs=[
            pl.BlockSpec((NPAD, D), lambda i: (0, 0)),
            pl.BlockSpec((_R, D), lambda i: (i, 0)),
            pl.BlockSpec((NPAD, D), lambda i: (0, 0)),
            pl.BlockSpec((1, D), lambda i: (0, 0)),
            pl.BlockSpec((D, D), lambda i: (0, 0)),
        ],
        out_specs=[
            pl.BlockSpec((_R, D), lambda i: (i, 0)),
            pl.BlockSpec((_R, D), lambda i: (i, 0)),
        ],
        out_shape=[
            jax.ShapeDtypeStruct((N_NODES, D), jnp.float32),
            jax.ShapeDtypeStruct((N_NODES, D), jnp.float32),
        ],
    )(Spv, h1, dgp, b1, W2)


def _tc_last(Spv, h2, dgp, b2, x1, Wlin, blin):
    def body(sp_ref, h_ref, dg_ref, b_ref, x1_ref, wl_ref, bl_ref, o_ref):
        dis = _dis_block(dg_ref)
        agg = _unpack(sp_ref, h_ref)
        x2 = jnp.maximum(agg * dis + b_ref[...], 0.0)
        hsum = x1_ref[...] + x2
        logits = jnp.dot(
            hsum, wl_ref[...], preferred_element_type=jnp.float32) + bl_ref[...]
        m = jnp.max(logits, axis=1, keepdims=True)
        lse = jnp.log(jnp.sum(jnp.exp(logits - m), axis=1, keepdims=True))
        o_ref[...] = logits - m - lse

    return pl.pallas_call(
        body,
        grid=(N_NODES // _R,),
        in_specs=[
            pl.BlockSpec((NPAD, D), lambda i: (0, 0)),
            pl.BlockSpec((_R, D), lambda i: (i, 0)),
            pl.BlockSpec((NPAD, D), lambda i: (0, 0)),
            pl.BlockSpec((1, D), lambda i: (0, 0)),
            pl.BlockSpec((_R, D), lambda i: (i, 0)),
            pl.BlockSpec((D, N_CLASSES), lambda i: (0, 0)),
            pl.BlockSpec((1, N_CLASSES), lambda i: (0, 0)),
        ],
        out_specs=pl.BlockSpec((_R, N_CLASSES), lambda i: (i, 0)),
        out_shape=jax.ShapeDtypeStruct((N_NODES, N_CLASSES), jnp.float32),
    )(Spv, h2, dgp, b2, x1, Wlin, blin)


def kernel(x, edge_index, W1, b1, W2, b2, Wlin, blin):
    ei = edge_index.astype(jnp.int32)
    n_extra = E_PAD - N_EDGES
    pad_rows = (jnp.arange(n_extra, dtype=jnp.int32) * 131) % N_NODES
    pad_cols = PAD_COL + (jnp.arange(n_extra, dtype=jnp.int32) % (NPAD - PAD_COL))
    row_p = jnp.concatenate([ei[0], pad_rows])
    col_p = jnp.concatenate([ei[1], pad_cols])
    col_deg = col_p.reshape(NW, NCH_DEG, CHUNK)
    rowlo3 = (2 * row_p).reshape(NS, NCH_SCAT, CHUNK)
    rowhi3 = (2 * row_p + 1).reshape(NS, NCH_SCAT, CHUNK)
    col3 = col_p.reshape(NS, NCH_SCAT, CHUNK)

    ones16 = jnp.ones((CHUNK, 16), jnp.float32)
    zeros16 = jnp.zeros((RPT, 16), jnp.float32)
    zerosH = jnp.zeros((RPT, DH), jnp.float32)

    dgp = _deg_partials(col_deg, ones16, zeros16)

    h1 = _tc_first(x, W1, dgp)
    S1 = _scatter_partials(h1.reshape(2 * N_NODES, DH), rowlo3, rowhi3, col3,
                           zerosH)
    x1, h2 = _tc_mid(S1, h1, dgp, b1.reshape(1, D), W2)
    S2 = _scatter_partials(h2.reshape(2 * N_NODES, DH), rowlo3, rowhi3, col3,
                           zerosH)
    return _tc_last(S2, h2, dgp, b2.reshape(1, D), x1,
                    Wlin, blin.reshape(1, N_CLASSES))
